# Initial kernel scaffold; baseline (speedup 1.0000x reference)
import jax
import jax.numpy as jnp
from jax.experimental import pallas as pl


def kernel(target, style_map):
    raise NotImplementedError("write your pallas kernel here")


# speedup vs baseline: 168986.1419x; 168986.1419x over previous
import jax
import jax.numpy as jnp
from jax.experimental import pallas as pl

_EPS = 1e-07


def _box3(x):
    r = x[:-2, :] + x[1:-1, :] + x[2:, :]
    return r[:, :-2] + r[:, 1:-1] + r[:, 2:]


def _matting_trace_kernel(t_ref, u_ref, out_ref):
    t = t_ref[...]
    u = u_ref[...]
    t0, t1, t2 = t[0], t[1], t[2]
    u0, u1, u2 = u[0], u[1], u[2]

    St = [_box3(t0), _box3(t1), _box3(t2)]
    s = [_box3(u0), _box3(u1), _box3(u2)]
    e_term = _box3(u0 * u0 + u1 * u1 + u2 * u2)

    ts = (t0, t1, t2)
    us = (u0, u1, u2)
    Q = {(c, d): _box3(ts[c] * ts[d]) for c in range(3) for d in range(c, 3)}
    R = [[_box3(ts[c] * us[d]) for d in range(3)] for c in range(3)]

    ninv = jnp.float32(1.0 / 9.0)
    mu = [St[c] * ninv for c in range(3)]

    def A(c, d):
        a = Q[(min(c, d), max(c, d))] * ninv - mu[c] * mu[d]
        if c == d:
            a = a + jnp.float32(_EPS / 9.0)
        return a

    m00, m01, m02 = A(0, 0), A(0, 1), A(0, 2)
    m11, m12, m22 = A(1, 1), A(1, 2), A(2, 2)
    c00 = m11 * m22 - m12 * m12
    c01 = m02 * m12 - m01 * m22
    c02 = m01 * m12 - m02 * m11
    c11 = m00 * m22 - m02 * m02
    c12 = m01 * m02 - m00 * m12
    c22 = m00 * m11 - m01 * m01
    det = m00 * c00 + m01 * c01 + m02 * c02

    B = [[R[c][d] - mu[c] * s[d] for d in range(3)] for c in range(3)]
    def G(c, d):
        return B[c][0] * B[d][0] + B[c][1] * B[d][1] + B[c][2] * B[d][2]

    quad = (c00 * G(0, 0) + c11 * G(1, 1) + c22 * G(2, 2)
            + 2.0 * (c01 * G(0, 1) + c02 * G(0, 2) + c12 * G(1, 2)))
    s_dot_s = s[0] * s[0] + s[1] * s[1] + s[2] * s[2]

    contrib = e_term - ninv * (s_dot_s + quad / det)
    out_ref[...] = jnp.sum(contrib, keepdims=True)


def kernel(target, style_map):
    out = pl.pallas_call(
        _matting_trace_kernel,
        out_shape=jax.ShapeDtypeStruct((1, 1), jnp.float32),
    )(target.astype(jnp.float32), style_map.astype(jnp.float32))
    return out[0, 0].astype(jnp.float64)
